# baseline (speedup 1.0000x reference)
import functools

import jax
import jax.numpy as jnp
from jax import lax
from jax.experimental import pallas as pl
from jax.experimental.pallas import tpu as pltpu
from jax.experimental.pallas import tpu_sc as plsc

N_NODES = 100000
D = 128
DW = D // 2
VROWS = 1025
NT = 513
NI = 256
K = 10
KP = 16

NC = 2
NS = 16
NW = NC * NS

B = 32
NSUB = N_NODES // B
FULL = NSUB // NW
EXTRA = NSUB - FULL * NW


def _tree_sum(vals):
    while len(vals) > 1:
        nxt = [vals[i] + vals[i + 1] for i in range(0, len(vals) - 1, 2)]
        if len(vals) % 2:
            nxt.append(vals[-1])
        vals = nxt
    return vals[0]


def _sc_body(table_hbm, idx_hbm, out_hbm, table_v, idx_v0, idx_v1,
             out_v0, out_v1, isem0, isem1, osem0, osem1):
    wid = lax.axis_index("s") * NC + lax.axis_index("c")
    nr = jnp.where(wid < EXTRA, FULL + 1, FULL)
    isems = (isem0, isem1)
    osems = (osem0, osem1)
    idx_vs = (idx_v0, idx_v1)
    out_vs = (out_v0, out_v1)

    def chunk(t):
        return wid + t * NW

    pltpu.sync_copy(table_hbm, table_v)

    def compute_write(t, p):
        base = chunk(t) * B

        @pl.when(t >= 2)
        def _():
            pltpu.make_async_copy(out_vs[p], out_hbm.at[pl.ds(base, B)],
                                  osems[p]).wait()

        def node_body(m, cc):
            for b in (2 * m, 2 * m + 1):
                iv = idx_vs[p][b, pl.ds(0, 16)]
                half = [(iv[k] >> 1, (iv[k] & 1) * DW) for k in range(K)]
                for j in range(DW // 16):
                    acc = _tree_sum([
                        plsc.bitcast(table_v[r, pl.ds(off + j * 16, 16)],
                                     jnp.bfloat16)
                        for r, off in half])
                    lo, hi = plsc.unpack(acc,
                                         format=plsc.PackFormat.INTERLEAVED)
                    out_vs[p][b, pl.ds(j * 32, 16)] = lo
                    out_vs[p][b, pl.ds(j * 32 + 16, 16)] = hi
            return cc

        lax.fori_loop(0, B // 2, node_body, 0)

        pltpu.async_copy(out_vs[p], out_hbm.at[pl.ds(base, B)], osems[p])

    def prefetch_idx(t, p):
        pltpu.async_copy(idx_hbm.at[chunk(t)], idx_vs[p], isems[p])

    def wait_idx(t, p):
        pltpu.make_async_copy(idx_hbm.at[chunk(t)], idx_vs[p],
                              isems[p]).wait()

    pltpu.sync_copy(idx_hbm.at[chunk(jnp.int32(0))], idx_v0)
    prefetch_idx(jnp.int32(1), 1)

    def body(v, carry):
        t0 = 2 * v
        t1 = t0 + 1
        t2 = t0 + 2
        t3 = t0 + 3
        compute_write(t0, 0)
        prefetch_idx(t2, 0)
        wait_idx(t1, 1)
        compute_write(t1, 1)

        @pl.when(t3 < nr)
        def _():
            prefetch_idx(t3, 1)

        wait_idx(t2, 0)
        return carry

    lax.fori_loop(0, FULL // 2, body, 0)

    compute_write(jnp.int32(FULL - 1), 0)

    @pl.when(nr > FULL)
    def _():
        wait_idx(jnp.int32(FULL), 1)
        compute_write(jnp.int32(FULL), 1)

    pltpu.make_async_copy(out_v0, out_hbm.at[pl.ds(0, B)], osem0).wait()
    pltpu.make_async_copy(out_v1, out_hbm.at[pl.ds(0, B)], osem1).wait()


def kernel(node_type, in_degree, out_degree, node_weight, in_degree_weight,
           out_degree_weight):
    table_f32 = jnp.concatenate(
        [node_weight, in_degree_weight, out_degree_weight], axis=0)
    table_bf = table_f32.astype(jnp.bfloat16)
    table_bf = table_bf.reshape(VROWS, 4, 2, 16).transpose(0, 1, 3, 2)
    table_bf = jnp.concatenate(
        [table_bf.reshape(VROWS, D),
         jnp.zeros((1, D), jnp.bfloat16)], axis=0)
    table = lax.bitcast_convert_type(
        table_bf.reshape(VROWS + 1, DW, 2), jnp.int32).reshape(
            (VROWS + 1) // 2, 2 * DW)

    idx = jnp.concatenate(
        [node_type.astype(jnp.int32),
         (in_degree.astype(jnp.int32) + NT)[:, None],
         (out_degree.astype(jnp.int32) + NT + NI)[:, None]], axis=1)
    idx = jnp.pad(idx, ((0, 0), (0, KP - K))).reshape(NSUB, B, KP)

    mesh = plsc.VectorSubcoreMesh(core_axis_name="c", subcore_axis_name="s",
                                  num_cores=NC, num_subcores=NS)
    run = functools.partial(
        pl.kernel,
        out_type=jax.ShapeDtypeStruct((N_NODES, D), jnp.float32),
        mesh=mesh,
        scratch_types=[
            pltpu.VMEM(((VROWS + 1) // 2, 2 * DW), jnp.int32),
            pltpu.VMEM((B, KP), jnp.int32),
            pltpu.VMEM((B, KP), jnp.int32),
            pltpu.VMEM((B, D), jnp.float32),
            pltpu.VMEM((B, D), jnp.float32),
            pltpu.SemaphoreType.DMA,
            pltpu.SemaphoreType.DMA,
            pltpu.SemaphoreType.DMA,
            pltpu.SemaphoreType.DMA,
        ],
        compiler_params=pltpu.CompilerParams(needs_layout_passes=False),
    )(_sc_body)
    return run(table, idx)


# speedup vs baseline: 1.0133x; 1.0133x over previous
import functools

import jax
import jax.numpy as jnp
from jax import lax
from jax.experimental import pallas as pl
from jax.experimental.pallas import tpu as pltpu
from jax.experimental.pallas import tpu_sc as plsc

N_NODES = 100000
D = 128
DW = D // 2
VROWS = 1025
NT = 513
NI = 256
K = 10
KP = 16

NC = 2
NS = 16
NW = NC * NS

B = 80
NSUB = N_NODES // B
FULL = NSUB // NW
EXTRA = NSUB - FULL * NW


def _tree_sum(vals):
    while len(vals) > 1:
        nxt = [vals[i] + vals[i + 1] for i in range(0, len(vals) - 1, 2)]
        if len(vals) % 2:
            nxt.append(vals[-1])
        vals = nxt
    return vals[0]


def _sc_body(table_hbm, idx_hbm, out_hbm, table_v, idx_v0, idx_v1,
             out_v0, out_v1, isem0, isem1, osem0, osem1):
    wid = lax.axis_index("s") * NC + lax.axis_index("c")
    nr = jnp.where(wid < EXTRA, FULL + 1, FULL)
    isems = (isem0, isem1)
    osems = (osem0, osem1)
    idx_vs = (idx_v0, idx_v1)
    out_vs = (out_v0, out_v1)

    def chunk(t):
        return wid + t * NW

    pltpu.sync_copy(table_hbm, table_v)

    def compute_write(t, p):
        base = chunk(t) * B

        @pl.when(t >= 2)
        def _():
            pltpu.make_async_copy(out_vs[p], out_hbm.at[pl.ds(base, B)],
                                  osems[p]).wait()

        def node_body(b, cc):
            iv = idx_vs[p][b, pl.ds(0, 16)]
            half = [(iv[k] >> 1, (iv[k] & 1) * DW) for k in range(K)]
            for j in range(DW // 16):
                acc = _tree_sum([
                    plsc.bitcast(table_v[r, pl.ds(off + j * 16, 16)],
                                 jnp.bfloat16)
                    for r, off in half])
                lo, hi = plsc.unpack(acc, format=plsc.PackFormat.INTERLEAVED)
                out_vs[p][b, pl.ds(j * 32, 16)] = lo
                out_vs[p][b, pl.ds(j * 32 + 16, 16)] = hi
            return cc

        lax.fori_loop(0, B, node_body, 0)

        pltpu.async_copy(out_vs[p], out_hbm.at[pl.ds(base, B)], osems[p])

    def prefetch_idx(t, p):
        pltpu.async_copy(idx_hbm.at[chunk(t)], idx_vs[p], isems[p])

    def wait_idx(t, p):
        pltpu.make_async_copy(idx_hbm.at[chunk(t)], idx_vs[p],
                              isems[p]).wait()

    pltpu.sync_copy(idx_hbm.at[chunk(jnp.int32(0))], idx_v0)
    prefetch_idx(jnp.int32(1), 1)

    def body(v, carry):
        t0 = 2 * v
        t1 = t0 + 1
        t2 = t0 + 2
        t3 = t0 + 3
        compute_write(t0, 0)
        prefetch_idx(t2, 0)
        wait_idx(t1, 1)
        compute_write(t1, 1)

        @pl.when(t3 < nr)
        def _():
            prefetch_idx(t3, 1)

        wait_idx(t2, 0)
        return carry

    lax.fori_loop(0, FULL // 2, body, 0)

    compute_write(jnp.int32(FULL - 1), 0)

    @pl.when(nr > FULL)
    def _():
        wait_idx(jnp.int32(FULL), 1)
        compute_write(jnp.int32(FULL), 1)

    pltpu.make_async_copy(out_v0, out_hbm.at[pl.ds(0, B)], osem0).wait()
    pltpu.make_async_copy(out_v1, out_hbm.at[pl.ds(0, B)], osem1).wait()


def kernel(node_type, in_degree, out_degree, node_weight, in_degree_weight,
           out_degree_weight):
    table_f32 = jnp.concatenate(
        [node_weight, in_degree_weight, out_degree_weight], axis=0)
    table_bf = table_f32.astype(jnp.bfloat16)
    table_bf = table_bf.reshape(VROWS, 4, 2, 16).transpose(0, 1, 3, 2)
    table_bf = jnp.concatenate(
        [table_bf.reshape(VROWS, D),
         jnp.zeros((1, D), jnp.bfloat16)], axis=0)
    table = lax.bitcast_convert_type(
        table_bf.reshape(VROWS + 1, DW, 2), jnp.int32).reshape(
            (VROWS + 1) // 2, 2 * DW)

    idx = jnp.concatenate(
        [node_type.astype(jnp.int32),
         (in_degree.astype(jnp.int32) + NT)[:, None],
         (out_degree.astype(jnp.int32) + NT + NI)[:, None]], axis=1)
    idx = jnp.pad(idx, ((0, 0), (0, KP - K))).reshape(NSUB, B, KP)

    mesh = plsc.VectorSubcoreMesh(core_axis_name="c", subcore_axis_name="s",
                                  num_cores=NC, num_subcores=NS)
    run = functools.partial(
        pl.kernel,
        out_type=jax.ShapeDtypeStruct((N_NODES, D), jnp.float32),
        mesh=mesh,
        scratch_types=[
            pltpu.VMEM(((VROWS + 1) // 2, 2 * DW), jnp.int32),
            pltpu.VMEM((B, KP), jnp.int32),
            pltpu.VMEM((B, KP), jnp.int32),
            pltpu.VMEM((B, D), jnp.float32),
            pltpu.VMEM((B, D), jnp.float32),
            pltpu.SemaphoreType.DMA,
            pltpu.SemaphoreType.DMA,
            pltpu.SemaphoreType.DMA,
            pltpu.SemaphoreType.DMA,
        ],
        compiler_params=pltpu.CompilerParams(needs_layout_passes=False),
    )(_sc_body)
    return run(table, idx)


# speedup vs baseline: 1.4532x; 1.4342x over previous
import functools

import jax
import jax.numpy as jnp
from jax import lax
from jax.experimental import pallas as pl
from jax.experimental.pallas import tpu as pltpu
from jax.experimental.pallas import tpu_sc as plsc

N_NODES = 100000
D = 128
DW = D // 2
VROWS = 1025
NT = 513
NI = 256
K = 10
KP = 16

NC = 2
NS = 16
NW = NC * NS

B = 32
NSUB = N_NODES // B
FULL = NSUB // NW
EXTRA = NSUB - FULL * NW


def _tree_sum(vals):
    while len(vals) > 1:
        nxt = [vals[i] + vals[i + 1] for i in range(0, len(vals) - 1, 2)]
        if len(vals) % 2:
            nxt.append(vals[-1])
        vals = nxt
    return vals[0]


def _sc_body(table_hbm, idx_hbm, out_hbm, table_v, idx_v0, idx_v1,
             out_v0, out_v1, isem0, isem1, osem0, osem1):
    wid = lax.axis_index("s") * NC + lax.axis_index("c")
    nr = jnp.where(wid < EXTRA, FULL + 1, FULL)
    isems = (isem0, isem1)
    osems = (osem0, osem1)
    idx_vs = (idx_v0, idx_v1)
    out_vs = (out_v0, out_v1)

    def chunk(t):
        return wid + t * NW

    pltpu.sync_copy(table_hbm, table_v)

    def compute_write(t, p):
        base = chunk(t) * B

        @pl.when(t >= 2)
        def _():
            pltpu.make_async_copy(out_vs[p], out_hbm.at[pl.ds(base, B)],
                                  osems[p]).wait()

        @plsc.parallel_loop(0, B, 1)
        def _node(b):
            iv = idx_vs[p][b, pl.ds(0, 16)]
            half = [(iv[k] >> 1, (iv[k] & 1) * DW) for k in range(K)]
            for j in range(DW // 16):
                acc = _tree_sum([
                    plsc.bitcast(table_v[r, pl.ds(off + j * 16, 16)],
                                 jnp.bfloat16)
                    for r, off in half])
                lo, hi = plsc.unpack(acc, format=plsc.PackFormat.INTERLEAVED)
                out_vs[p][b, pl.ds(j * 32, 16)] = lo
                out_vs[p][b, pl.ds(j * 32 + 16, 16)] = hi

        pltpu.async_copy(out_vs[p], out_hbm.at[pl.ds(base, B)], osems[p])

    def prefetch_idx(t, p):
        pltpu.async_copy(idx_hbm.at[chunk(t)], idx_vs[p], isems[p])

    def wait_idx(t, p):
        pltpu.make_async_copy(idx_hbm.at[chunk(t)], idx_vs[p],
                              isems[p]).wait()

    pltpu.sync_copy(idx_hbm.at[chunk(jnp.int32(0))], idx_v0)
    prefetch_idx(jnp.int32(1), 1)

    def body(v, carry):
        t0 = 2 * v
        t1 = t0 + 1
        t2 = t0 + 2
        t3 = t0 + 3
        compute_write(t0, 0)
        prefetch_idx(t2, 0)
        wait_idx(t1, 1)
        compute_write(t1, 1)

        @pl.when(t3 < nr)
        def _():
            prefetch_idx(t3, 1)

        wait_idx(t2, 0)
        return carry

    lax.fori_loop(0, FULL // 2, body, 0)

    compute_write(jnp.int32(FULL - 1), 0)

    @pl.when(nr > FULL)
    def _():
        wait_idx(jnp.int32(FULL), 1)
        compute_write(jnp.int32(FULL), 1)

    pltpu.make_async_copy(out_v0, out_hbm.at[pl.ds(0, B)], osem0).wait()
    pltpu.make_async_copy(out_v1, out_hbm.at[pl.ds(0, B)], osem1).wait()


def kernel(node_type, in_degree, out_degree, node_weight, in_degree_weight,
           out_degree_weight):
    table_f32 = jnp.concatenate(
        [node_weight, in_degree_weight, out_degree_weight], axis=0)
    table_bf = table_f32.astype(jnp.bfloat16)
    table_bf = table_bf.reshape(VROWS, 4, 2, 16).transpose(0, 1, 3, 2)
    table_bf = jnp.concatenate(
        [table_bf.reshape(VROWS, D),
         jnp.zeros((1, D), jnp.bfloat16)], axis=0)
    table = lax.bitcast_convert_type(
        table_bf.reshape(VROWS + 1, DW, 2), jnp.int32).reshape(
            (VROWS + 1) // 2, 2 * DW)

    idx = jnp.concatenate(
        [node_type.astype(jnp.int32),
         (in_degree.astype(jnp.int32) + NT)[:, None],
         (out_degree.astype(jnp.int32) + NT + NI)[:, None]], axis=1)
    idx = jnp.pad(idx, ((0, 0), (0, KP - K))).reshape(NSUB, B, KP)

    mesh = plsc.VectorSubcoreMesh(core_axis_name="c", subcore_axis_name="s",
                                  num_cores=NC, num_subcores=NS)
    run = functools.partial(
        pl.kernel,
        out_type=jax.ShapeDtypeStruct((N_NODES, D), jnp.float32),
        mesh=mesh,
        scratch_types=[
            pltpu.VMEM(((VROWS + 1) // 2, 2 * DW), jnp.int32),
            pltpu.VMEM((B, KP), jnp.int32),
            pltpu.VMEM((B, KP), jnp.int32),
            pltpu.VMEM((B, D), jnp.float32),
            pltpu.VMEM((B, D), jnp.float32),
            pltpu.SemaphoreType.DMA,
            pltpu.SemaphoreType.DMA,
            pltpu.SemaphoreType.DMA,
            pltpu.SemaphoreType.DMA,
        ],
        compiler_params=pltpu.CompilerParams(needs_layout_passes=False),
    )(_sc_body)
    return run(table, idx)
